# Initial kernel scaffold; baseline (speedup 1.0000x reference)
import jax
import jax.numpy as jnp
from jax.experimental import pallas as pl


def kernel(flatCDK, CWK, CK, flat_eta, N_per_word, flatZ, doc_indexes, time_ind_per_word, flatW):
    raise NotImplementedError("write your pallas kernel here")


# speedup vs baseline: 5.8914x; 5.8914x over previous
import functools

import jax
import jax.numpy as jnp
from jax import lax
from jax.experimental import pallas as pl
from jax.experimental.pallas import tpu as pltpu
from jax.experimental.pallas import tpu_sc as plsc

T = 8
V = 50000
K = 64
D_TOTAL = 20000
ALL_WORDS = 1000000

NC = 2
NS = 16
NW = NC * NS

DOCS_PER_TILE = D_TOTAL // NW
CDK_BLK = DOCS_PER_TILE * K
CWK_BLK = (V * K) // NW
CHUNK1 = 8192
CHUNK2 = 8192

_ALPHA = 50.0 / K
_ETA_NUM = 1.0 + _ALPHA
_ETA_DEN = K * _ALPHA


def _wid():
    return lax.axis_index("s") * NC + lax.axis_index("c")


def _docs_body(d_hbm, z_hbm, t_hbm, n_hbm, cdk_in, eta_in, s_hbm,
               cdk_out, eta_out, ckp_out,
               bounds_v, cdk_v, eta_v, ck_v, idx_v,
               dbuf, zbuf, tbuf, nbuf, ck_sh):
    wid = _wid()
    cid = lax.axis_index("c")
    sid = lax.axis_index("s")

    pltpu.sync_copy(s_hbm, bounds_v)
    cdk_off = pl.multiple_of(wid * CDK_BLK, 8)
    pltpu.sync_copy(cdk_in.at[pl.ds(cdk_off, CDK_BLK)], cdk_v)
    pltpu.sync_copy(eta_in.at[pl.ds(cdk_off, CDK_BLK)], eta_v)

    zeros16 = jnp.zeros((16,), jnp.float32)
    for r in range(16):
        for cdiv in range(K // 16):
            ck_v[r, pl.ds(cdiv * 16, 16)] = zeros16
    idx_v[...] = lax.iota(jnp.int32, 16)

    @pl.when(sid == 0)
    def _():
        pltpu.sync_copy(ck_v, ck_sh)

    plsc.subcore_barrier()

    bv = bounds_v[pl.ds(wid, 16)]
    lo = bv[0]
    hi = bv[1]
    start = lo - lax.rem(lo, 8)
    nchunks = lax.div(hi - start + (CHUNK1 - 1), CHUNK1)

    ones = jnp.ones((16,), jnp.float32)
    lane = lax.iota(jnp.int32, 16)
    base_rel = wid * CDK_BLK

    def chunk_body(c, _):
        ustart = start + c * CHUNK1
        base = pl.multiple_of(jnp.minimum(ustart, ALL_WORDS - CHUNK1), 8)
        pltpu.sync_copy(d_hbm.at[pl.ds(base, CHUNK1)], dbuf)
        pltpu.sync_copy(z_hbm.at[pl.ds(base, CHUNK1)], zbuf)
        pltpu.sync_copy(t_hbm.at[pl.ds(base, CHUNK1)], tbuf)
        pltpu.sync_copy(n_hbm.at[pl.ds(base, CHUNK1)], nbuf)
        lmax = jnp.maximum(lo, ustart)

        def group_body(j, _):
            for u in range(4):
                x = (j * 4 + u) * 16
                pos = base + x + lane
                m = (pos >= lmax) & (pos < hi)
                dd = dbuf[pl.ds(x, 16)]
                zz = zbuf[pl.ds(x, 16)]
                tt = tbuf[pl.ds(x, 16)]
                nn = nbuf[pl.ds(x, 16)]
                rel = dd * K + zz - base_rel
                plsc.addupdate_scatter(cdk_v, [rel], ones, mask=m)
                val = _ETA_NUM / (nn + _ETA_DEN)
                plsc.addupdate_scatter(eta_v, [rel], val, mask=m)
                plsc.addupdate_scatter(ck_v, [tt, zz], ones, mask=m)
            return 0

        lax.fori_loop(0, CHUNK1 // 64, group_body, 0)
        return 0

    lax.fori_loop(0, nchunks, chunk_body, 0)

    pltpu.sync_copy(cdk_v, cdk_out.at[pl.ds(cdk_off, CDK_BLK)])
    pltpu.sync_copy(eta_v, eta_out.at[pl.ds(cdk_off, CDK_BLK)])

    pltpu.sync_copy(ck_v, ck_sh.at[idx_v], add=True)
    plsc.subcore_barrier()

    @pl.when(sid == 0)
    def _():
        pltpu.sync_copy(ck_sh, ckp_out.at[cid])


def _cwk_body(w_hbm, z_hbm, cwk_in, e_hbm,
              cwk_out,
              bounds_v, tab_v, wbuf, zbuf):
    wid = _wid()
    pltpu.sync_copy(e_hbm, bounds_v)

    ones = jnp.ones((16,), jnp.float32)
    lane = lax.iota(jnp.int32, 16)
    blk = wid * CWK_BLK
    ev = bounds_v[pl.ds(0, 16)]

    for t in range(T):
        off = pl.multiple_of(t * (V * K) + blk, 8)
        pltpu.sync_copy(cwk_in.at[pl.ds(off, CWK_BLK)], tab_v)

        lo = ev[t]
        hi = ev[t + 1]
        start = lo - lax.rem(lo, 8)
        nchunks = lax.div(hi - start + (CHUNK2 - 1), CHUNK2)
        stagger = lax.div(nchunks * wid, NW)

        def chunk_body(c, _):
            cc = lax.rem(c + stagger, nchunks)
            ustart = start + cc * CHUNK2
            base = pl.multiple_of(jnp.minimum(ustart, ALL_WORDS - CHUNK2), 8)
            pltpu.sync_copy(w_hbm.at[pl.ds(base, CHUNK2)], wbuf)
            pltpu.sync_copy(z_hbm.at[pl.ds(base, CHUNK2)], zbuf)
            lmax = jnp.maximum(lo, ustart)

            def group_body(j, _):
                for u in range(4):
                    x = (j * 4 + u) * 16
                    pos = base + x + lane
                    ww = wbuf[pl.ds(x, 16)]
                    zz = zbuf[pl.ds(x, 16)]
                    idx = ww * K + zz - blk
                    m = ((pos >= lmax) & (pos < hi)
                         & (idx >= 0) & (idx < CWK_BLK))
                    plsc.addupdate_scatter(tab_v, [idx], ones, mask=m)
                return 0

            lax.fori_loop(0, CHUNK2 // 64, group_body, 0)
            return 0

        lax.fori_loop(0, nchunks, chunk_body, 0)

        pltpu.sync_copy(tab_v, cwk_out.at[pl.ds(off, CWK_BLK)])


def kernel(flatCDK, CWK, CK, flat_eta, N_per_word, flatZ, doc_indexes,
           time_ind_per_word, flatW):
    d32 = doc_indexes.astype(jnp.int32)
    z32 = flatZ.astype(jnp.int32)
    t32 = time_ind_per_word.astype(jnp.int32)
    w32 = flatW.astype(jnp.int32)

    doc_bounds = jnp.arange(0, D_TOTAL + 1, DOCS_PER_TILE, dtype=jnp.int32)
    s_arr = jnp.searchsorted(d32, doc_bounds).astype(jnp.int32)
    s_arr = jnp.concatenate([s_arr, jnp.zeros((15,), jnp.int32)])

    t_bounds = jnp.arange(0, T + 1, dtype=jnp.int32)
    e_arr = jnp.searchsorted(t32, t_bounds).astype(jnp.int32)
    e_arr = jnp.concatenate([e_arr, jnp.zeros((7,), jnp.int32)])

    mesh = plsc.VectorSubcoreMesh(core_axis_name="c", subcore_axis_name="s",
                                  num_cores=NC, num_subcores=NS)

    docs_call = pl.kernel(
        _docs_body,
        out_type=(
            jax.ShapeDtypeStruct((D_TOTAL * K,), jnp.float32),
            jax.ShapeDtypeStruct((D_TOTAL * K,), jnp.float32),
            jax.ShapeDtypeStruct((NC, 16, K), jnp.float32),
        ),
        mesh=mesh,
        compiler_params=pltpu.CompilerParams(needs_layout_passes=False),
        scratch_types=(
            pltpu.VMEM((48,), jnp.int32),
            pltpu.VMEM((CDK_BLK,), jnp.float32),
            pltpu.VMEM((CDK_BLK,), jnp.float32),
            pltpu.VMEM((16, K), jnp.float32),
            pltpu.VMEM((16,), jnp.int32),
            pltpu.VMEM((CHUNK1,), jnp.int32),
            pltpu.VMEM((CHUNK1,), jnp.int32),
            pltpu.VMEM((CHUNK1,), jnp.int32),
            pltpu.VMEM((CHUNK1,), jnp.float32),
            pltpu.VMEM_SHARED((16, K), jnp.float32),
        ),
    )
    cdk_out, eta_out, ckp = docs_call(
        d32, z32, t32, N_per_word.astype(jnp.float32),
        flatCDK.reshape(-1), flat_eta.reshape(-1), s_arr)

    cwk_call = pl.kernel(
        _cwk_body,
        out_type=jax.ShapeDtypeStruct((T * V * K,), jnp.float32),
        mesh=mesh,
        compiler_params=pltpu.CompilerParams(needs_layout_passes=False),
        scratch_types=(
            pltpu.VMEM((16,), jnp.int32),
            pltpu.VMEM((CWK_BLK,), jnp.float32),
            pltpu.VMEM((CHUNK2,), jnp.int32),
            pltpu.VMEM((CHUNK2,), jnp.int32),
        ),
    )
    cwk_out = cwk_call(w32, z32, CWK.reshape(-1), e_arr)

    ck_out = CK + ckp[0, :T, :] + ckp[1, :T, :]

    return (cdk_out.reshape(D_TOTAL, K), cwk_out.reshape(T, V, K), ck_out,
            eta_out.reshape(D_TOTAL, K))


# speedup vs baseline: 6.7006x; 1.1374x over previous
import jax
import jax.numpy as jnp
from jax import lax
from jax.experimental import pallas as pl
from jax.experimental.pallas import tpu as pltpu
from jax.experimental.pallas import tpu_sc as plsc

T = 8
V = 50000
K = 64
D_TOTAL = 20000
ALL_WORDS = 1000000

NC = 2
NS = 16
NW = NC * NS

DOCS_PER_TILE = D_TOTAL // NW
CDK_BLK = DOCS_PER_TILE * K
CWK_BLK = (V * K) // NW
CHUNK1 = 4096
CHUNK2 = 8192

_ALPHA = 50.0 / K
_ETA_NUM = 1.0 + _ALPHA
_ETA_DEN = K * _ALPHA


def _wid():
    return lax.axis_index("s") * NC + lax.axis_index("c")


def _docs_body(d_hbm, z_hbm, t_hbm, n_hbm, w_hbm, cdk_in, eta_in, s_hbm,
               cdk_out, eta_out, ckp_out, pk_out,
               bounds_v, cdk_v, eta_v, ck_v, idx_v,
               dbuf, zbuf, tbuf, nbuf, wbuf, pkbuf, ck_sh):
    wid = _wid()
    cid = lax.axis_index("c")
    sid = lax.axis_index("s")

    pltpu.sync_copy(s_hbm, bounds_v)
    cdk_off = pl.multiple_of(wid * CDK_BLK, 8)
    pltpu.sync_copy(cdk_in.at[pl.ds(cdk_off, CDK_BLK)], cdk_v)
    pltpu.sync_copy(eta_in.at[pl.ds(cdk_off, CDK_BLK)], eta_v)

    zeros16 = jnp.zeros((16,), jnp.float32)
    for r in range(16):
        for cdiv in range(K // 16):
            ck_v[r, pl.ds(cdiv * 16, 16)] = zeros16
    idx_v[...] = lax.iota(jnp.int32, 16)

    @pl.when(sid == 0)
    def _():
        pltpu.sync_copy(ck_v, ck_sh)

    plsc.subcore_barrier()

    bv = bounds_v[pl.ds(wid, 16)]
    lo = bv[0]
    hi = bv[1]
    start = lo - lax.rem(lo, 8)
    nchunks = lax.div(hi - start + (CHUNK1 - 1), CHUNK1)

    ones = jnp.ones((16,), jnp.float32)
    lane = lax.iota(jnp.int32, 16)
    base_rel = wid * CDK_BLK

    def chunk_body(c, _):
        ustart = start + c * CHUNK1
        base = pl.multiple_of(jnp.minimum(ustart, ALL_WORDS - CHUNK1), 8)
        pltpu.sync_copy(d_hbm.at[pl.ds(base, CHUNK1)], dbuf)
        pltpu.sync_copy(z_hbm.at[pl.ds(base, CHUNK1)], zbuf)
        pltpu.sync_copy(t_hbm.at[pl.ds(base, CHUNK1)], tbuf)
        pltpu.sync_copy(n_hbm.at[pl.ds(base, CHUNK1)], nbuf)
        pltpu.sync_copy(w_hbm.at[pl.ds(base, CHUNK1)], wbuf)
        lmax = jnp.maximum(lo, ustart)

        def group_body(j, _):
            for u in range(4):
                x = (j * 4 + u) * 16
                pos = base + x + lane
                m = (pos >= lmax) & (pos < hi)
                dd = dbuf[pl.ds(x, 16)]
                zz = zbuf[pl.ds(x, 16)]
                tt = tbuf[pl.ds(x, 16)]
                nn = nbuf[pl.ds(x, 16)]
                ww = wbuf[pl.ds(x, 16)]
                pkbuf[pl.ds(x, 16)] = ww * K + zz
                rel = dd * K + zz - base_rel
                plsc.addupdate_scatter(cdk_v, [rel], ones, mask=m)
                val = _ETA_NUM / (nn + _ETA_DEN)
                plsc.addupdate_scatter(eta_v, [rel], val, mask=m)
                plsc.addupdate_scatter(ck_v, [tt, zz], ones, mask=m)
            return 0

        lax.fori_loop(0, CHUNK1 // 64, group_body, 0)
        pltpu.sync_copy(pkbuf, pk_out.at[pl.ds(base, CHUNK1)])
        return 0

    lax.fori_loop(0, nchunks, chunk_body, 0)

    pltpu.sync_copy(cdk_v, cdk_out.at[pl.ds(cdk_off, CDK_BLK)])
    pltpu.sync_copy(eta_v, eta_out.at[pl.ds(cdk_off, CDK_BLK)])

    pltpu.sync_copy(ck_v, ck_sh.at[idx_v], add=True)
    plsc.subcore_barrier()

    @pl.when(sid == 0)
    def _():
        pltpu.sync_copy(ck_sh, ckp_out.at[cid])


def _cwk_body(pk_hbm, cwk_in, e_hbm,
              cwk_out,
              bounds_v, tab_v, pbuf):
    wid = _wid()
    pltpu.sync_copy(e_hbm, bounds_v)

    ones = jnp.ones((16,), jnp.float32)
    lane = lax.iota(jnp.int32, 16)
    blk = wid * CWK_BLK
    ublk = jnp.uint32(CWK_BLK)
    ev = bounds_v[pl.ds(0, 16)]

    def edge(a, b):
        @pl.when(a < b)
        def _():
            astart = a - lax.rem(a, 8)
            base = pl.multiple_of(
                jnp.minimum(astart, ALL_WORDS - CHUNK2), 8)
            pltpu.sync_copy(pk_hbm.at[pl.ds(base, CHUNK2)], pbuf)

            def group_body(j, _):
                for u in range(4):
                    x = (j * 4 + u) * 16
                    pos = base + x + lane
                    pk = pbuf[pl.ds(x, 16)]
                    local = pk - blk
                    m = ((pos >= a) & (pos < b)
                         & (plsc.bitcast(local, jnp.uint32) < ublk))
                    plsc.addupdate_scatter(tab_v, [local], ones, mask=m)
                return 0

            lax.fori_loop(0, CHUNK2 // 64, group_body, 0)

    for t in range(T):
        off = pl.multiple_of(t * (V * K) + blk, 8)
        pltpu.sync_copy(cwk_in.at[pl.ds(off, CWK_BLK)], tab_v)

        lo = ev[t]
        hi = ev[t + 1]
        start = lo - lax.rem(lo, 8)
        c0 = jnp.where(lo == start, 0, 1)
        nfe = lax.div(hi - start, CHUNK2)
        nmain = jnp.maximum(nfe - c0, 0)
        f0 = start + c0 * CHUNK2
        f1 = start + nfe * CHUNK2
        stagger = lax.div(nmain * wid, NW)

        def chunk_body(c, _):
            cc = c0 + lax.rem(c + stagger, jnp.maximum(nmain, 1))
            base = pl.multiple_of(start + cc * CHUNK2, 8)
            pltpu.sync_copy(pk_hbm.at[pl.ds(base, CHUNK2)], pbuf)

            def group_body(j, _):
                for u in range(16):
                    x = (j * 16 + u) * 16
                    pk = pbuf[pl.ds(x, 16)]
                    local = pk - blk
                    m = plsc.bitcast(local, jnp.uint32) < ublk
                    plsc.addupdate_scatter(tab_v, [local], ones, mask=m)
                return 0

            lax.fori_loop(0, CHUNK2 // 256, group_body, 0)
            return 0

        lax.fori_loop(0, nmain, chunk_body, 0)

        e1b = jnp.minimum(f0, hi)
        edge(lo, e1b)
        edge(jnp.maximum(f1, e1b), hi)

        pltpu.sync_copy(tab_v, cwk_out.at[pl.ds(off, CWK_BLK)])


def kernel(flatCDK, CWK, CK, flat_eta, N_per_word, flatZ, doc_indexes,
           time_ind_per_word, flatW):
    d32 = doc_indexes.astype(jnp.int32)
    z32 = flatZ.astype(jnp.int32)
    t32 = time_ind_per_word.astype(jnp.int32)
    w32 = flatW.astype(jnp.int32)

    doc_bounds = jnp.arange(0, D_TOTAL + 1, DOCS_PER_TILE, dtype=jnp.int32)
    s_arr = jnp.searchsorted(d32, doc_bounds).astype(jnp.int32)
    s_arr = jnp.concatenate([s_arr, jnp.zeros((15,), jnp.int32)])

    t_bounds = jnp.arange(0, T + 1, dtype=jnp.int32)
    e_arr = jnp.searchsorted(t32, t_bounds).astype(jnp.int32)
    e_arr = jnp.concatenate([e_arr, jnp.zeros((7,), jnp.int32)])

    mesh = plsc.VectorSubcoreMesh(core_axis_name="c", subcore_axis_name="s",
                                  num_cores=NC, num_subcores=NS)

    docs_call = pl.kernel(
        _docs_body,
        out_type=(
            jax.ShapeDtypeStruct((D_TOTAL * K,), jnp.float32),
            jax.ShapeDtypeStruct((D_TOTAL * K,), jnp.float32),
            jax.ShapeDtypeStruct((NC, 16, K), jnp.float32),
            jax.ShapeDtypeStruct((ALL_WORDS,), jnp.int32),
        ),
        mesh=mesh,
        compiler_params=pltpu.CompilerParams(needs_layout_passes=False),
        scratch_types=(
            pltpu.VMEM((48,), jnp.int32),
            pltpu.VMEM((CDK_BLK,), jnp.float32),
            pltpu.VMEM((CDK_BLK,), jnp.float32),
            pltpu.VMEM((16, K), jnp.float32),
            pltpu.VMEM((16,), jnp.int32),
            pltpu.VMEM((CHUNK1,), jnp.int32),
            pltpu.VMEM((CHUNK1,), jnp.int32),
            pltpu.VMEM((CHUNK1,), jnp.int32),
            pltpu.VMEM((CHUNK1,), jnp.float32),
            pltpu.VMEM((CHUNK1,), jnp.int32),
            pltpu.VMEM((CHUNK1,), jnp.int32),
            pltpu.VMEM_SHARED((16, K), jnp.float32),
        ),
    )
    cdk_out, eta_out, ckp, pk = docs_call(
        d32, z32, t32, N_per_word.astype(jnp.float32), w32,
        flatCDK.reshape(-1), flat_eta.reshape(-1), s_arr)

    cwk_call = pl.kernel(
        _cwk_body,
        out_type=jax.ShapeDtypeStruct((T * V * K,), jnp.float32),
        mesh=mesh,
        compiler_params=pltpu.CompilerParams(needs_layout_passes=False),
        scratch_types=(
            pltpu.VMEM((16,), jnp.int32),
            pltpu.VMEM((CWK_BLK,), jnp.float32),
            pltpu.VMEM((CHUNK2,), jnp.int32),
        ),
    )
    cwk_out = cwk_call(pk, CWK.reshape(-1), e_arr)

    ck_out = CK + ckp[0, :T, :] + ckp[1, :T, :]

    return (cdk_out.reshape(D_TOTAL, K), cwk_out.reshape(T, V, K), ck_out,
            eta_out.reshape(D_TOTAL, K))


# speedup vs baseline: 7.2319x; 1.0793x over previous
import jax
import jax.numpy as jnp
from jax import lax
from jax.experimental import pallas as pl
from jax.experimental.pallas import tpu as pltpu
from jax.experimental.pallas import tpu_sc as plsc

T = 8
V = 50000
K = 64
D_TOTAL = 20000
ALL_WORDS = 1000000

NC = 2
NS = 16
NW = NC * NS

DOCS_PER_TILE = D_TOTAL // NW
CDK_BLK = DOCS_PER_TILE * K
CWK_BLK = (V * K) // NW
CHUNK1 = 4096
CHUNK2 = 8192

_ALPHA = 50.0 / K
_ETA_NUM = 1.0 + _ALPHA
_ETA_DEN = K * _ALPHA


def _wid():
    return lax.axis_index("s") * NC + lax.axis_index("c")


def _docs_body(d_hbm, z_hbm, t_hbm, n_hbm, w_hbm, cdk_in, eta_in, s_hbm,
               cdk_out, eta_out, ckp_out, pk_out,
               bounds_v, cdk_v, eta_v, ck_v, idx_v,
               dbuf, zbuf, tbuf, nbuf, wbuf, pkbuf, ck_sh):
    wid = _wid()
    cid = lax.axis_index("c")
    sid = lax.axis_index("s")

    pltpu.sync_copy(s_hbm, bounds_v)
    cdk_off = pl.multiple_of(wid * CDK_BLK, 8)
    pltpu.sync_copy(cdk_in.at[pl.ds(cdk_off, CDK_BLK)], cdk_v)
    pltpu.sync_copy(eta_in.at[pl.ds(cdk_off, CDK_BLK)], eta_v)

    zeros16 = jnp.zeros((16,), jnp.float32)
    for r in range(16):
        for cdiv in range(K // 16):
            ck_v[r, pl.ds(cdiv * 16, 16)] = zeros16
    idx_v[...] = lax.iota(jnp.int32, 16)

    @pl.when(sid == 0)
    def _():
        pltpu.sync_copy(ck_v, ck_sh)

    plsc.subcore_barrier()

    bv = bounds_v[pl.ds(wid, 16)]
    lo = bv[0]
    hi = bv[1]
    start = lo - lax.rem(lo, 8)
    nchunks = lax.div(hi - start + (CHUNK1 - 1), CHUNK1)

    ones = jnp.ones((16,), jnp.float32)
    lane = lax.iota(jnp.int32, 16)
    base_rel = wid * CDK_BLK

    def chunk_body(c, _):
        ustart = start + c * CHUNK1
        base = pl.multiple_of(jnp.minimum(ustart, ALL_WORDS - CHUNK1), 8)
        pltpu.sync_copy(d_hbm.at[pl.ds(base, CHUNK1)], dbuf)
        pltpu.sync_copy(z_hbm.at[pl.ds(base, CHUNK1)], zbuf)
        pltpu.sync_copy(t_hbm.at[pl.ds(base, CHUNK1)], tbuf)
        pltpu.sync_copy(n_hbm.at[pl.ds(base, CHUNK1)], nbuf)
        pltpu.sync_copy(w_hbm.at[pl.ds(base, CHUNK1)], wbuf)
        lmax = jnp.maximum(lo, ustart)

        def group_body(j, _):
            for u in range(4):
                x = (j * 4 + u) * 16
                pos = base + x + lane
                m = (pos >= lmax) & (pos < hi)
                dd = dbuf[pl.ds(x, 16)]
                zz = zbuf[pl.ds(x, 16)]
                tt = tbuf[pl.ds(x, 16)]
                nn = nbuf[pl.ds(x, 16)]
                ww = wbuf[pl.ds(x, 16)]
                pkbuf[pl.ds(x, 16)] = ww * K + zz
                rel = dd * K + zz - base_rel
                plsc.addupdate_scatter(cdk_v, [rel], ones, mask=m)
                val = _ETA_NUM / (nn + _ETA_DEN)
                plsc.addupdate_scatter(eta_v, [rel], val, mask=m)
                plsc.addupdate_scatter(ck_v, [tt, zz], ones, mask=m)
            return 0

        lax.fori_loop(0, CHUNK1 // 64, group_body, 0)
        pltpu.sync_copy(pkbuf, pk_out.at[pl.ds(base, CHUNK1)])
        return 0

    lax.fori_loop(0, nchunks, chunk_body, 0)

    pltpu.sync_copy(cdk_v, cdk_out.at[pl.ds(cdk_off, CDK_BLK)])
    pltpu.sync_copy(eta_v, eta_out.at[pl.ds(cdk_off, CDK_BLK)])

    pltpu.sync_copy(ck_v, ck_sh.at[idx_v], add=True)
    plsc.subcore_barrier()

    @pl.when(sid == 0)
    def _():
        pltpu.sync_copy(ck_sh, ckp_out.at[cid])


def _cwk_body(pk_hbm, cwk_in, e_hbm,
              cwk_out,
              bounds_v, tab_v, pbuf, pbuf1, sem0, sem1):
    wid = _wid()
    pltpu.sync_copy(e_hbm, bounds_v)

    ones = jnp.ones((16,), jnp.float32)
    lane = lax.iota(jnp.int32, 16)
    blk = wid * CWK_BLK
    ublk = jnp.uint32(CWK_BLK)
    ev = bounds_v[pl.ds(0, 16)]

    def edge(a, b):
        @pl.when(a < b)
        def _():
            astart = a - lax.rem(a, 8)
            base = pl.multiple_of(
                jnp.minimum(astart, ALL_WORDS - CHUNK2), 8)
            pltpu.sync_copy(pk_hbm.at[pl.ds(base, CHUNK2)], pbuf)

            def group_body(j, _):
                for u in range(4):
                    x = (j * 4 + u) * 16
                    pos = base + x + lane
                    pk = pbuf[pl.ds(x, 16)]
                    local = pk - blk
                    m = ((pos >= a) & (pos < b)
                         & (plsc.bitcast(local, jnp.uint32) < ublk))
                    plsc.addupdate_scatter(tab_v, [local], ones, mask=m)
                return 0

            lax.fori_loop(0, CHUNK2 // 64, group_body, 0)

    for t in range(T):
        off = pl.multiple_of(t * (V * K) + blk, 8)
        pltpu.sync_copy(cwk_in.at[pl.ds(off, CWK_BLK)], tab_v)

        lo = ev[t]
        hi = ev[t + 1]
        start = lo - lax.rem(lo, 8)
        c0 = jnp.where(lo == start, 0, 1)
        nfe = lax.div(hi - start, CHUNK2)
        nmain = jnp.maximum(nfe - c0, 0)
        f0 = start + c0 * CHUNK2
        f1 = start + nfe * CHUNK2
        stagger = lax.div(nmain * wid, NW)

        def chunk_base(c):
            cc = c0 + lax.rem(c + stagger, jnp.maximum(nmain, 1))
            return pl.multiple_of(start + cc * CHUNK2, 8)

        def start_load(c, buf, sem):
            pltpu.async_copy(pk_hbm.at[pl.ds(chunk_base(c), CHUNK2)],
                             buf, sem)

        def wait_load(buf, sem):
            pltpu.make_async_copy(pk_hbm.at[pl.ds(0, CHUNK2)], buf,
                                  sem).wait()

        def compute(buf):
            def group_body(j, _):
                for u in range(16):
                    x = (j * 16 + u) * 16
                    pk = buf[pl.ds(x, 16)]
                    local = pk - blk
                    m = plsc.bitcast(local, jnp.uint32) < ublk
                    plsc.addupdate_scatter(tab_v, [local], ones, mask=m)
                return 0

            lax.fori_loop(0, CHUNK2 // 256, group_body, 0)

        @pl.when(nmain > 0)
        def _():
            start_load(0, pbuf, sem0)

        def pair_body(p, _):
            c_e = 2 * p
            wait_load(pbuf, sem0)

            @pl.when(c_e + 1 < nmain)
            def _():
                start_load(c_e + 1, pbuf1, sem1)

            compute(pbuf)

            @pl.when(c_e + 1 < nmain)
            def _():
                wait_load(pbuf1, sem1)

                @pl.when(c_e + 2 < nmain)
                def _():
                    start_load(c_e + 2, pbuf, sem0)

                compute(pbuf1)

            return 0

        lax.fori_loop(0, lax.div(nmain + 1, 2), pair_body, 0)

        e1b = jnp.minimum(f0, hi)
        edge(lo, e1b)
        edge(jnp.maximum(f1, e1b), hi)

        pltpu.sync_copy(tab_v, cwk_out.at[pl.ds(off, CWK_BLK)])


def kernel(flatCDK, CWK, CK, flat_eta, N_per_word, flatZ, doc_indexes,
           time_ind_per_word, flatW):
    d32 = doc_indexes.astype(jnp.int32)
    z32 = flatZ.astype(jnp.int32)
    t32 = time_ind_per_word.astype(jnp.int32)
    w32 = flatW.astype(jnp.int32)

    doc_bounds = jnp.arange(0, D_TOTAL + 1, DOCS_PER_TILE, dtype=jnp.int32)
    s_arr = jnp.searchsorted(d32, doc_bounds).astype(jnp.int32)
    s_arr = jnp.concatenate([s_arr, jnp.zeros((15,), jnp.int32)])

    t_bounds = jnp.arange(0, T + 1, dtype=jnp.int32)
    e_arr = jnp.searchsorted(t32, t_bounds).astype(jnp.int32)
    e_arr = jnp.concatenate([e_arr, jnp.zeros((7,), jnp.int32)])

    mesh = plsc.VectorSubcoreMesh(core_axis_name="c", subcore_axis_name="s",
                                  num_cores=NC, num_subcores=NS)

    docs_call = pl.kernel(
        _docs_body,
        out_type=(
            jax.ShapeDtypeStruct((D_TOTAL * K,), jnp.float32),
            jax.ShapeDtypeStruct((D_TOTAL * K,), jnp.float32),
            jax.ShapeDtypeStruct((NC, 16, K), jnp.float32),
            jax.ShapeDtypeStruct((ALL_WORDS,), jnp.int32),
        ),
        mesh=mesh,
        compiler_params=pltpu.CompilerParams(needs_layout_passes=False),
        scratch_types=(
            pltpu.VMEM((48,), jnp.int32),
            pltpu.VMEM((CDK_BLK,), jnp.float32),
            pltpu.VMEM((CDK_BLK,), jnp.float32),
            pltpu.VMEM((16, K), jnp.float32),
            pltpu.VMEM((16,), jnp.int32),
            pltpu.VMEM((CHUNK1,), jnp.int32),
            pltpu.VMEM((CHUNK1,), jnp.int32),
            pltpu.VMEM((CHUNK1,), jnp.int32),
            pltpu.VMEM((CHUNK1,), jnp.float32),
            pltpu.VMEM((CHUNK1,), jnp.int32),
            pltpu.VMEM((CHUNK1,), jnp.int32),
            pltpu.VMEM_SHARED((16, K), jnp.float32),
        ),
    )
    cdk_out, eta_out, ckp, pk = docs_call(
        d32, z32, t32, N_per_word.astype(jnp.float32), w32,
        flatCDK.reshape(-1), flat_eta.reshape(-1), s_arr)

    cwk_call = pl.kernel(
        _cwk_body,
        out_type=jax.ShapeDtypeStruct((T * V * K,), jnp.float32),
        mesh=mesh,
        compiler_params=pltpu.CompilerParams(needs_layout_passes=False),
        scratch_types=(
            pltpu.VMEM((16,), jnp.int32),
            pltpu.VMEM((CWK_BLK,), jnp.float32),
            pltpu.VMEM((CHUNK2,), jnp.int32),
            pltpu.VMEM((CHUNK2,), jnp.int32),
            pltpu.SemaphoreType.DMA,
            pltpu.SemaphoreType.DMA,
        ),
    )
    cwk_out = cwk_call(pk, CWK.reshape(-1), e_arr)

    ck_out = CK + ckp[0, :T, :] + ckp[1, :T, :]

    return (cdk_out.reshape(D_TOTAL, K), cwk_out.reshape(T, V, K), ck_out,
            eta_out.reshape(D_TOTAL, K))


# speedup vs baseline: 10.0521x; 1.3900x over previous
import jax
import jax.numpy as jnp
from jax import lax
from jax.experimental import pallas as pl
from jax.experimental.pallas import tpu as pltpu
from jax.experimental.pallas import tpu_sc as plsc

T = 8
V = 50000
K = 64
D_TOTAL = 20000
ALL_WORDS = 1000000

NC = 2
NS = 16
NW = NC * NS

DOCS_PER_TILE = D_TOTAL // NW
CDK_BLK = DOCS_PER_TILE * K
CWK_BLK = (V * K) // NW
CHUNK1 = 4096
CHUNK2 = 8192

_ALPHA = 50.0 / K
_ETA_NUM = 1.0 + _ALPHA
_ETA_DEN = K * _ALPHA


def _wid():
    return lax.axis_index("s") * NC + lax.axis_index("c")


def _docs_body(d_hbm, z_hbm, t_hbm, n_hbm, w_hbm, cdk_in, eta_in, s_hbm,
               cdk_out, eta_out, ckp_out, pk_out,
               bounds_v, cdk_v, eta_v, ck_v, idx_v,
               dbuf, zbuf, tbuf, nbuf, wbuf, pkbuf, ck_sh):
    wid = _wid()
    cid = lax.axis_index("c")
    sid = lax.axis_index("s")

    pltpu.sync_copy(s_hbm, bounds_v)
    cdk_off = pl.multiple_of(wid * CDK_BLK, 8)
    pltpu.sync_copy(cdk_in.at[pl.ds(cdk_off, CDK_BLK)], cdk_v)
    pltpu.sync_copy(eta_in.at[pl.ds(cdk_off, CDK_BLK)], eta_v)

    zeros16 = jnp.zeros((16,), jnp.float32)
    for r in range(16):
        for cdiv in range(K // 16):
            ck_v[r, pl.ds(cdiv * 16, 16)] = zeros16
    idx_v[...] = lax.iota(jnp.int32, 16)

    @pl.when(sid == 0)
    def _():
        pltpu.sync_copy(ck_v, ck_sh)

    plsc.subcore_barrier()

    bv = bounds_v[pl.ds(wid, 16)]
    lo = bv[0]
    hi = bv[1]
    start = lo - lax.rem(lo, 8)
    nchunks = lax.div(hi - start + (CHUNK1 - 1), CHUNK1)

    ones = jnp.ones((16,), jnp.float32)
    lane = lax.iota(jnp.int32, 16)
    base_rel = wid * CDK_BLK

    def chunk_body(c, _):
        ustart = start + c * CHUNK1
        base = pl.multiple_of(jnp.minimum(ustart, ALL_WORDS - CHUNK1), 8)
        pltpu.sync_copy(d_hbm.at[pl.ds(base, CHUNK1)], dbuf)
        pltpu.sync_copy(z_hbm.at[pl.ds(base, CHUNK1)], zbuf)
        pltpu.sync_copy(t_hbm.at[pl.ds(base, CHUNK1)], tbuf)
        pltpu.sync_copy(n_hbm.at[pl.ds(base, CHUNK1)], nbuf)
        pltpu.sync_copy(w_hbm.at[pl.ds(base, CHUNK1)], wbuf)
        lmax = jnp.maximum(lo, ustart)

        def group_body(j, _):
            for u in range(4):
                x = (j * 4 + u) * 16
                pos = base + x + lane
                m = (pos >= lmax) & (pos < hi)
                dd = dbuf[pl.ds(x, 16)]
                zz = zbuf[pl.ds(x, 16)]
                tt = tbuf[pl.ds(x, 16)]
                nn = nbuf[pl.ds(x, 16)]
                ww = wbuf[pl.ds(x, 16)]
                pkbuf[pl.ds(x, 16)] = ww * K + zz
                rel = dd * K + zz - base_rel
                plsc.addupdate_scatter(cdk_v, [rel], ones, mask=m)
                val = _ETA_NUM / (nn + _ETA_DEN)
                plsc.addupdate_scatter(eta_v, [rel], val, mask=m)
                plsc.addupdate_scatter(ck_v, [tt, zz], ones, mask=m)
            return 0

        lax.fori_loop(0, CHUNK1 // 64, group_body, 0)
        pltpu.sync_copy(pkbuf, pk_out.at[pl.ds(base, CHUNK1)])
        return 0

    lax.fori_loop(0, nchunks, chunk_body, 0)

    pltpu.sync_copy(cdk_v, cdk_out.at[pl.ds(cdk_off, CDK_BLK)])
    pltpu.sync_copy(eta_v, eta_out.at[pl.ds(cdk_off, CDK_BLK)])

    pltpu.sync_copy(ck_v, ck_sh.at[idx_v], add=True)
    plsc.subcore_barrier()

    @pl.when(sid == 0)
    def _():
        pltpu.sync_copy(ck_sh, ckp_out.at[cid])


def _cwk_body(pk_hbm, cwk_in, e_hbm,
              cwk_out,
              bounds_v, tab_v, pbuf, pbuf1, sem0, sem1):
    wid = _wid()
    pltpu.sync_copy(e_hbm, bounds_v)

    ones = jnp.ones((16,), jnp.float32)
    lane = lax.iota(jnp.int32, 16)
    blk = wid * CWK_BLK
    ublk = jnp.uint32(CWK_BLK)
    ev = bounds_v[pl.ds(0, 16)]

    def edge(a, b):
        @pl.when(a < b)
        def _():
            astart = a - lax.rem(a, 8)
            base = pl.multiple_of(
                jnp.minimum(astart, ALL_WORDS - CHUNK2), 8)
            pltpu.sync_copy(pk_hbm.at[pl.ds(base, CHUNK2)], pbuf)

            def group_body(j, _):
                for u in range(4):
                    x = (j * 4 + u) * 16
                    pos = base + x + lane
                    pk = pbuf[pl.ds(x, 16)]
                    local = pk - blk
                    m = ((pos >= a) & (pos < b)
                         & (plsc.bitcast(local, jnp.uint32) < ublk))
                    plsc.addupdate_scatter(tab_v, [local], ones, mask=m)
                return 0

            lax.fori_loop(0, CHUNK2 // 64, group_body, 0)

    for t in range(T):
        off = pl.multiple_of(t * (V * K) + blk, 8)
        pltpu.sync_copy(cwk_in.at[pl.ds(off, CWK_BLK)], tab_v)

        lo = ev[t]
        hi = ev[t + 1]
        start = lo - lax.rem(lo, 8)
        c0 = jnp.where(lo == start, 0, 1)
        nfe = lax.div(hi - start, CHUNK2)
        nmain = jnp.maximum(nfe - c0, 0)
        f0 = start + c0 * CHUNK2
        f1 = start + nfe * CHUNK2
        stagger = lax.div(nmain * wid, NW)

        def chunk_base(c):
            cc = c0 + lax.rem(c + stagger, jnp.maximum(nmain, 1))
            return pl.multiple_of(start + cc * CHUNK2, 8)

        def start_load(c, buf, sem):
            pltpu.async_copy(pk_hbm.at[pl.ds(chunk_base(c), CHUNK2)],
                             buf, sem)

        def wait_load(buf, sem):
            pltpu.make_async_copy(pk_hbm.at[pl.ds(0, CHUNK2)], buf,
                                  sem).wait()

        def compute(buf):
            @plsc.parallel_loop(0, CHUNK2 // 16, 1, unroll=8)
            def group_body(g):
                x = g * 16
                pk = buf[pl.ds(x, 16)]
                local = pk - blk
                m = plsc.bitcast(local, jnp.uint32) < ublk
                plsc.addupdate_scatter(tab_v, [local], ones, mask=m)

        @pl.when(nmain > 0)
        def _():
            start_load(0, pbuf, sem0)

        def pair_body(p, _):
            c_e = 2 * p
            wait_load(pbuf, sem0)

            @pl.when(c_e + 1 < nmain)
            def _():
                start_load(c_e + 1, pbuf1, sem1)

            compute(pbuf)

            @pl.when(c_e + 1 < nmain)
            def _():
                wait_load(pbuf1, sem1)

                @pl.when(c_e + 2 < nmain)
                def _():
                    start_load(c_e + 2, pbuf, sem0)

                compute(pbuf1)

            return 0

        lax.fori_loop(0, lax.div(nmain + 1, 2), pair_body, 0)

        e1b = jnp.minimum(f0, hi)
        edge(lo, e1b)
        edge(jnp.maximum(f1, e1b), hi)

        pltpu.sync_copy(tab_v, cwk_out.at[pl.ds(off, CWK_BLK)])


def kernel(flatCDK, CWK, CK, flat_eta, N_per_word, flatZ, doc_indexes,
           time_ind_per_word, flatW):
    d32 = doc_indexes.astype(jnp.int32)
    z32 = flatZ.astype(jnp.int32)
    t32 = time_ind_per_word.astype(jnp.int32)
    w32 = flatW.astype(jnp.int32)

    doc_bounds = jnp.arange(0, D_TOTAL + 1, DOCS_PER_TILE, dtype=jnp.int32)
    s_arr = jnp.searchsorted(d32, doc_bounds).astype(jnp.int32)
    s_arr = jnp.concatenate([s_arr, jnp.zeros((15,), jnp.int32)])

    t_bounds = jnp.arange(0, T + 1, dtype=jnp.int32)
    e_arr = jnp.searchsorted(t32, t_bounds).astype(jnp.int32)
    e_arr = jnp.concatenate([e_arr, jnp.zeros((7,), jnp.int32)])

    mesh = plsc.VectorSubcoreMesh(core_axis_name="c", subcore_axis_name="s",
                                  num_cores=NC, num_subcores=NS)

    docs_call = pl.kernel(
        _docs_body,
        out_type=(
            jax.ShapeDtypeStruct((D_TOTAL * K,), jnp.float32),
            jax.ShapeDtypeStruct((D_TOTAL * K,), jnp.float32),
            jax.ShapeDtypeStruct((NC, 16, K), jnp.float32),
            jax.ShapeDtypeStruct((ALL_WORDS,), jnp.int32),
        ),
        mesh=mesh,
        compiler_params=pltpu.CompilerParams(needs_layout_passes=False),
        scratch_types=(
            pltpu.VMEM((48,), jnp.int32),
            pltpu.VMEM((CDK_BLK,), jnp.float32),
            pltpu.VMEM((CDK_BLK,), jnp.float32),
            pltpu.VMEM((16, K), jnp.float32),
            pltpu.VMEM((16,), jnp.int32),
            pltpu.VMEM((CHUNK1,), jnp.int32),
            pltpu.VMEM((CHUNK1,), jnp.int32),
            pltpu.VMEM((CHUNK1,), jnp.int32),
            pltpu.VMEM((CHUNK1,), jnp.float32),
            pltpu.VMEM((CHUNK1,), jnp.int32),
            pltpu.VMEM((CHUNK1,), jnp.int32),
            pltpu.VMEM_SHARED((16, K), jnp.float32),
        ),
    )
    cdk_out, eta_out, ckp, pk = docs_call(
        d32, z32, t32, N_per_word.astype(jnp.float32), w32,
        flatCDK.reshape(-1), flat_eta.reshape(-1), s_arr)

    cwk_call = pl.kernel(
        _cwk_body,
        out_type=jax.ShapeDtypeStruct((T * V * K,), jnp.float32),
        mesh=mesh,
        compiler_params=pltpu.CompilerParams(needs_layout_passes=False),
        scratch_types=(
            pltpu.VMEM((16,), jnp.int32),
            pltpu.VMEM((CWK_BLK,), jnp.float32),
            pltpu.VMEM((CHUNK2,), jnp.int32),
            pltpu.VMEM((CHUNK2,), jnp.int32),
            pltpu.SemaphoreType.DMA,
            pltpu.SemaphoreType.DMA,
        ),
    )
    cwk_out = cwk_call(pk, CWK.reshape(-1), e_arr)

    ck_out = CK + ckp[0, :T, :] + ckp[1, :T, :]

    return (cdk_out.reshape(D_TOTAL, K), cwk_out.reshape(T, V, K), ck_out,
            eta_out.reshape(D_TOTAL, K))


# speedup vs baseline: 10.4787x; 1.0424x over previous
import jax
import jax.numpy as jnp
from jax import lax
from jax.experimental import pallas as pl
from jax.experimental.pallas import tpu as pltpu
from jax.experimental.pallas import tpu_sc as plsc

T = 8
V = 50000
K = 64
D_TOTAL = 20000
ALL_WORDS = 1000000

NC = 2
NS = 16
NW = NC * NS

DOCS_PER_TILE = D_TOTAL // NW
CDK_BLK = DOCS_PER_TILE * K
CWK_BLK = (V * K) // NW
CHUNK1 = 4096
CHUNK2 = 8192

_ALPHA = 50.0 / K
_ETA_NUM = 1.0 + _ALPHA
_ETA_DEN = K * _ALPHA


def _wid():
    return lax.axis_index("s") * NC + lax.axis_index("c")


def _docs_body(d_hbm, z_hbm, t_hbm, n_hbm, w_hbm, cdk_in, eta_in, s_hbm,
               cdk_out, eta_out, ckp_out, pk_out,
               bounds_v, cdk_v, eta_v, ck_v, idx_v,
               dbuf, zbuf, tbuf, nbuf, wbuf, pkbuf, ck_sh):
    wid = _wid()
    cid = lax.axis_index("c")
    sid = lax.axis_index("s")

    pltpu.sync_copy(s_hbm, bounds_v)
    cdk_off = pl.multiple_of(wid * CDK_BLK, 8)
    pltpu.sync_copy(cdk_in.at[pl.ds(cdk_off, CDK_BLK)], cdk_v)
    pltpu.sync_copy(eta_in.at[pl.ds(cdk_off, CDK_BLK)], eta_v)

    zeros16 = jnp.zeros((16,), jnp.float32)
    for r in range(16):
        for cdiv in range(K // 16):
            ck_v[r, pl.ds(cdiv * 16, 16)] = zeros16
    idx_v[...] = lax.iota(jnp.int32, 16)

    @pl.when(sid == 0)
    def _():
        pltpu.sync_copy(ck_v, ck_sh)

    plsc.subcore_barrier()

    bv = bounds_v[pl.ds(wid, 16)]
    lo = bv[0]
    hi = bv[1]
    start = lo - lax.rem(lo, 8)
    nchunks = lax.div(hi - start + (CHUNK1 - 1), CHUNK1)

    ones = jnp.ones((16,), jnp.float32)
    lane = lax.iota(jnp.int32, 16)
    base_rel = wid * CDK_BLK

    def chunk_body(c, _):
        ustart = start + c * CHUNK1
        base = pl.multiple_of(jnp.minimum(ustart, ALL_WORDS - CHUNK1), 8)
        pltpu.sync_copy(d_hbm.at[pl.ds(base, CHUNK1)], dbuf)
        pltpu.sync_copy(z_hbm.at[pl.ds(base, CHUNK1)], zbuf)
        pltpu.sync_copy(t_hbm.at[pl.ds(base, CHUNK1)], tbuf)
        pltpu.sync_copy(n_hbm.at[pl.ds(base, CHUNK1)], nbuf)
        pltpu.sync_copy(w_hbm.at[pl.ds(base, CHUNK1)], wbuf)
        lmax = jnp.maximum(lo, ustart)

        @plsc.parallel_loop(0, CHUNK1 // 16, 1, unroll=4)
        def group_body(g):
            x = g * 16
            pos = base + x + lane
            m = (pos >= lmax) & (pos < hi)
            dd = dbuf[pl.ds(x, 16)]
            zz = zbuf[pl.ds(x, 16)]
            tt = tbuf[pl.ds(x, 16)]
            nn = nbuf[pl.ds(x, 16)]
            ww = wbuf[pl.ds(x, 16)]
            pkbuf[pl.ds(x, 16)] = ww * K + zz
            rel = dd * K + zz - base_rel
            plsc.addupdate_scatter(cdk_v, [rel], ones, mask=m)
            val = _ETA_NUM / (nn + _ETA_DEN)
            plsc.addupdate_scatter(eta_v, [rel], val, mask=m)
            plsc.addupdate_scatter(ck_v, [tt, zz], ones, mask=m)
        pltpu.sync_copy(pkbuf, pk_out.at[pl.ds(base, CHUNK1)])
        return 0

    lax.fori_loop(0, nchunks, chunk_body, 0)

    pltpu.sync_copy(cdk_v, cdk_out.at[pl.ds(cdk_off, CDK_BLK)])
    pltpu.sync_copy(eta_v, eta_out.at[pl.ds(cdk_off, CDK_BLK)])

    pltpu.sync_copy(ck_v, ck_sh.at[idx_v], add=True)
    plsc.subcore_barrier()

    @pl.when(sid == 0)
    def _():
        pltpu.sync_copy(ck_sh, ckp_out.at[cid])


def _cwk_body(pk_hbm, cwk_in, e_hbm,
              cwk_out,
              bounds_v, tab_v, pbuf, pbuf1, sem0, sem1):
    wid = _wid()
    pltpu.sync_copy(e_hbm, bounds_v)

    ones = jnp.ones((16,), jnp.float32)
    lane = lax.iota(jnp.int32, 16)
    blk = wid * CWK_BLK
    ublk = jnp.uint32(CWK_BLK)
    ev = bounds_v[pl.ds(0, 16)]

    def edge(a, b):
        @pl.when(a < b)
        def _():
            astart = a - lax.rem(a, 8)
            base = pl.multiple_of(
                jnp.minimum(astart, ALL_WORDS - CHUNK2), 8)
            pltpu.sync_copy(pk_hbm.at[pl.ds(base, CHUNK2)], pbuf)

            @plsc.parallel_loop(0, CHUNK2 // 16, 1, unroll=4)
            def group_body(g):
                x = g * 16
                pos = base + x + lane
                pk = pbuf[pl.ds(x, 16)]
                local = pk - blk
                m = ((pos >= a) & (pos < b)
                     & (plsc.bitcast(local, jnp.uint32) < ublk))
                plsc.addupdate_scatter(tab_v, [local], ones, mask=m)

    for t in range(T):
        off = pl.multiple_of(t * (V * K) + blk, 8)
        pltpu.sync_copy(cwk_in.at[pl.ds(off, CWK_BLK)], tab_v)

        lo = ev[t]
        hi = ev[t + 1]
        start = lo - lax.rem(lo, 8)
        c0 = jnp.where(lo == start, 0, 1)
        nfe = lax.div(hi - start, CHUNK2)
        nmain = jnp.maximum(nfe - c0, 0)
        f0 = start + c0 * CHUNK2
        f1 = start + nfe * CHUNK2
        stagger = lax.div(nmain * wid, NW)

        def chunk_base(c):
            cc = c0 + lax.rem(c + stagger, jnp.maximum(nmain, 1))
            return pl.multiple_of(start + cc * CHUNK2, 8)

        def start_load(c, buf, sem):
            pltpu.async_copy(pk_hbm.at[pl.ds(chunk_base(c), CHUNK2)],
                             buf, sem)

        def wait_load(buf, sem):
            pltpu.make_async_copy(pk_hbm.at[pl.ds(0, CHUNK2)], buf,
                                  sem).wait()

        def compute(buf):
            @plsc.parallel_loop(0, CHUNK2 // 16, 1, unroll=8)
            def group_body(g):
                x = g * 16
                pk = buf[pl.ds(x, 16)]
                local = pk - blk
                m = plsc.bitcast(local, jnp.uint32) < ublk
                plsc.addupdate_scatter(tab_v, [local], ones, mask=m)

        @pl.when(nmain > 0)
        def _():
            start_load(0, pbuf, sem0)

        def pair_body(p, _):
            c_e = 2 * p
            wait_load(pbuf, sem0)

            @pl.when(c_e + 1 < nmain)
            def _():
                start_load(c_e + 1, pbuf1, sem1)

            compute(pbuf)

            @pl.when(c_e + 1 < nmain)
            def _():
                wait_load(pbuf1, sem1)

                @pl.when(c_e + 2 < nmain)
                def _():
                    start_load(c_e + 2, pbuf, sem0)

                compute(pbuf1)

            return 0

        lax.fori_loop(0, lax.div(nmain + 1, 2), pair_body, 0)

        e1b = jnp.minimum(f0, hi)
        edge(lo, e1b)
        edge(jnp.maximum(f1, e1b), hi)

        pltpu.sync_copy(tab_v, cwk_out.at[pl.ds(off, CWK_BLK)])


def kernel(flatCDK, CWK, CK, flat_eta, N_per_word, flatZ, doc_indexes,
           time_ind_per_word, flatW):
    d32 = doc_indexes.astype(jnp.int32)
    z32 = flatZ.astype(jnp.int32)
    t32 = time_ind_per_word.astype(jnp.int32)
    w32 = flatW.astype(jnp.int32)

    doc_bounds = jnp.arange(0, D_TOTAL + 1, DOCS_PER_TILE, dtype=jnp.int32)
    s_arr = jnp.searchsorted(d32, doc_bounds).astype(jnp.int32)
    s_arr = jnp.concatenate([s_arr, jnp.zeros((15,), jnp.int32)])

    t_bounds = jnp.arange(0, T + 1, dtype=jnp.int32)
    e_arr = jnp.searchsorted(t32, t_bounds).astype(jnp.int32)
    e_arr = jnp.concatenate([e_arr, jnp.zeros((7,), jnp.int32)])

    mesh = plsc.VectorSubcoreMesh(core_axis_name="c", subcore_axis_name="s",
                                  num_cores=NC, num_subcores=NS)

    docs_call = pl.kernel(
        _docs_body,
        out_type=(
            jax.ShapeDtypeStruct((D_TOTAL * K,), jnp.float32),
            jax.ShapeDtypeStruct((D_TOTAL * K,), jnp.float32),
            jax.ShapeDtypeStruct((NC, 16, K), jnp.float32),
            jax.ShapeDtypeStruct((ALL_WORDS,), jnp.int32),
        ),
        mesh=mesh,
        compiler_params=pltpu.CompilerParams(needs_layout_passes=False),
        scratch_types=(
            pltpu.VMEM((48,), jnp.int32),
            pltpu.VMEM((CDK_BLK,), jnp.float32),
            pltpu.VMEM((CDK_BLK,), jnp.float32),
            pltpu.VMEM((16, K), jnp.float32),
            pltpu.VMEM((16,), jnp.int32),
            pltpu.VMEM((CHUNK1,), jnp.int32),
            pltpu.VMEM((CHUNK1,), jnp.int32),
            pltpu.VMEM((CHUNK1,), jnp.int32),
            pltpu.VMEM((CHUNK1,), jnp.float32),
            pltpu.VMEM((CHUNK1,), jnp.int32),
            pltpu.VMEM((CHUNK1,), jnp.int32),
            pltpu.VMEM_SHARED((16, K), jnp.float32),
        ),
    )
    cdk_out, eta_out, ckp, pk = docs_call(
        d32, z32, t32, N_per_word.astype(jnp.float32), w32,
        flatCDK.reshape(-1), flat_eta.reshape(-1), s_arr)

    cwk_call = pl.kernel(
        _cwk_body,
        out_type=jax.ShapeDtypeStruct((T * V * K,), jnp.float32),
        mesh=mesh,
        compiler_params=pltpu.CompilerParams(needs_layout_passes=False),
        scratch_types=(
            pltpu.VMEM((16,), jnp.int32),
            pltpu.VMEM((CWK_BLK,), jnp.float32),
            pltpu.VMEM((CHUNK2,), jnp.int32),
            pltpu.VMEM((CHUNK2,), jnp.int32),
            pltpu.SemaphoreType.DMA,
            pltpu.SemaphoreType.DMA,
        ),
    )
    cwk_out = cwk_call(pk, CWK.reshape(-1), e_arr)

    ck_out = CK + ckp[0, :T, :] + ckp[1, :T, :]

    return (cdk_out.reshape(D_TOTAL, K), cwk_out.reshape(T, V, K), ck_out,
            eta_out.reshape(D_TOTAL, K))


# speedup vs baseline: 11.6205x; 1.1090x over previous
import jax
import jax.numpy as jnp
from jax import lax
from jax.experimental import pallas as pl
from jax.experimental.pallas import tpu as pltpu
from jax.experimental.pallas import tpu_sc as plsc

T = 8
V = 50000
K = 64
D_TOTAL = 20000
ALL_WORDS = 1000000

NC = 2
NS = 16
NW = NC * NS

DOCS_PER_TILE = D_TOTAL // NW
CDK_BLK = DOCS_PER_TILE * K
CWK_BLK = (V * K) // NW
CHUNK1 = 4096
CHUNK2 = 8192

_ALPHA = 50.0 / K
_ETA_NUM = 1.0 + _ALPHA
_ETA_DEN = K * _ALPHA


def _wid():
    return lax.axis_index("s") * NC + lax.axis_index("c")


def _docs_body(d_hbm, z_hbm, t_hbm, n_hbm, w_hbm, s_hbm,
               cdk_out, eta_out, ckp_out, pk_out,
               bounds_v, cdk_v, eta_v, ck_v, idx_v,
               dbuf, zbuf, tbuf, nbuf, wbuf, pkbuf, ck_sh):
    wid = _wid()
    cid = lax.axis_index("c")
    sid = lax.axis_index("s")

    pltpu.sync_copy(s_hbm, bounds_v)
    cdk_off = pl.multiple_of(wid * CDK_BLK, 8)

    zeros16 = jnp.zeros((16,), jnp.float32)

    @plsc.parallel_loop(0, CDK_BLK // 16, 1, unroll=8)
    def _(g):
        cdk_v[pl.ds(g * 16, 16)] = zeros16
        eta_v[pl.ds(g * 16, 16)] = zeros16

    for r in range(16):
        for cdiv in range(K // 16):
            ck_v[r, pl.ds(cdiv * 16, 16)] = zeros16
    idx_v[...] = lax.iota(jnp.int32, 16)

    @pl.when(sid == 0)
    def _():
        pltpu.sync_copy(ck_v, ck_sh)

    plsc.subcore_barrier()

    bv = bounds_v[pl.ds(wid, 16)]
    lo = bv[0]
    hi = bv[1]
    start = lo - lax.rem(lo, 8)
    nchunks = lax.div(hi - start + (CHUNK1 - 1), CHUNK1)

    ones = jnp.ones((16,), jnp.float32)
    lane = lax.iota(jnp.int32, 16)
    base_rel = wid * CDK_BLK

    def chunk_body(c, _):
        ustart = start + c * CHUNK1
        base = pl.multiple_of(jnp.minimum(ustart, ALL_WORDS - CHUNK1), 8)
        pltpu.sync_copy(d_hbm.at[pl.ds(base, CHUNK1)], dbuf)
        pltpu.sync_copy(z_hbm.at[pl.ds(base, CHUNK1)], zbuf)
        pltpu.sync_copy(t_hbm.at[pl.ds(base, CHUNK1)], tbuf)
        pltpu.sync_copy(n_hbm.at[pl.ds(base, CHUNK1)], nbuf)
        pltpu.sync_copy(w_hbm.at[pl.ds(base, CHUNK1)], wbuf)
        lmax = jnp.maximum(lo, ustart)

        @plsc.parallel_loop(0, CHUNK1 // 16, 1, unroll=4)
        def group_body(g):
            x = g * 16
            pos = base + x + lane
            m = (pos >= lmax) & (pos < hi)
            dd = dbuf[pl.ds(x, 16)]
            zz = zbuf[pl.ds(x, 16)]
            tt = tbuf[pl.ds(x, 16)]
            nn = nbuf[pl.ds(x, 16)]
            ww = wbuf[pl.ds(x, 16)]
            pkbuf[pl.ds(x, 16)] = ww * K + zz
            rel = dd * K + zz - base_rel
            plsc.addupdate_scatter(cdk_v, [rel], ones, mask=m)
            val = _ETA_NUM / (nn + _ETA_DEN)
            plsc.addupdate_scatter(eta_v, [rel], val, mask=m)
            plsc.addupdate_scatter(ck_v, [tt, zz], ones, mask=m)
        pltpu.sync_copy(pkbuf, pk_out.at[pl.ds(base, CHUNK1)])
        return 0

    lax.fori_loop(0, nchunks, chunk_body, 0)

    pltpu.sync_copy(cdk_v, cdk_out.at[pl.ds(cdk_off, CDK_BLK)])
    pltpu.sync_copy(eta_v, eta_out.at[pl.ds(cdk_off, CDK_BLK)])

    pltpu.sync_copy(ck_v, ck_sh.at[idx_v], add=True)
    plsc.subcore_barrier()

    @pl.when(sid == 0)
    def _():
        pltpu.sync_copy(ck_sh, ckp_out.at[cid])


def _cwk_body(pk_hbm, e_hbm,
              cwk_out,
              bounds_v, tab_v, pbuf, pbuf1, sem0, sem1):
    wid = _wid()
    pltpu.sync_copy(e_hbm, bounds_v)

    ones = jnp.ones((16,), jnp.float32)
    zeros16 = jnp.zeros((16,), jnp.float32)
    lane = lax.iota(jnp.int32, 16)
    blk = wid * CWK_BLK
    ublk = jnp.uint32(CWK_BLK)
    ev = bounds_v[pl.ds(0, 16)]

    def edge(a, b):
        @pl.when(a < b)
        def _():
            astart = a - lax.rem(a, 8)
            base = pl.multiple_of(
                jnp.minimum(astart, ALL_WORDS - CHUNK2), 8)
            pltpu.sync_copy(pk_hbm.at[pl.ds(base, CHUNK2)], pbuf)

            @plsc.parallel_loop(0, CHUNK2 // 16, 1, unroll=4)
            def group_body(g):
                x = g * 16
                pos = base + x + lane
                pk = pbuf[pl.ds(x, 16)]
                local = pk - blk
                m = ((pos >= a) & (pos < b)
                     & (plsc.bitcast(local, jnp.uint32) < ublk))
                plsc.addupdate_scatter(tab_v, [local], ones, mask=m)

    for t in range(T):
        off = pl.multiple_of(t * (V * K) + blk, 8)

        @plsc.parallel_loop(0, CWK_BLK // 16, 1, unroll=8)
        def _(g):
            tab_v[pl.ds(g * 16, 16)] = zeros16

        lo = ev[t]
        hi = ev[t + 1]
        start = lo - lax.rem(lo, 8)
        c0 = jnp.where(lo == start, 0, 1)
        nfe = lax.div(hi - start, CHUNK2)
        nmain = jnp.maximum(nfe - c0, 0)
        f0 = start + c0 * CHUNK2
        f1 = start + nfe * CHUNK2
        stagger = lax.div(nmain * wid, NW)

        def chunk_base(c):
            cc = c0 + lax.rem(c + stagger, jnp.maximum(nmain, 1))
            return pl.multiple_of(start + cc * CHUNK2, 8)

        def start_load(c, buf, sem):
            pltpu.async_copy(pk_hbm.at[pl.ds(chunk_base(c), CHUNK2)],
                             buf, sem)

        def wait_load(buf, sem):
            pltpu.make_async_copy(pk_hbm.at[pl.ds(0, CHUNK2)], buf,
                                  sem).wait()

        def compute(buf):
            @plsc.parallel_loop(0, CHUNK2 // 16, 1, unroll=8)
            def group_body(g):
                x = g * 16
                pk = buf[pl.ds(x, 16)]
                local = pk - blk
                m = plsc.bitcast(local, jnp.uint32) < ublk
                plsc.addupdate_scatter(tab_v, [local], ones, mask=m)

        @pl.when(nmain > 0)
        def _():
            start_load(0, pbuf, sem0)

        def pair_body(p, _):
            c_e = 2 * p
            wait_load(pbuf, sem0)

            @pl.when(c_e + 1 < nmain)
            def _():
                start_load(c_e + 1, pbuf1, sem1)

            compute(pbuf)

            @pl.when(c_e + 1 < nmain)
            def _():
                wait_load(pbuf1, sem1)

                @pl.when(c_e + 2 < nmain)
                def _():
                    start_load(c_e + 2, pbuf, sem0)

                compute(pbuf1)

            return 0

        lax.fori_loop(0, lax.div(nmain + 1, 2), pair_body, 0)

        e1b = jnp.minimum(f0, hi)
        edge(lo, e1b)
        edge(jnp.maximum(f1, e1b), hi)

        pltpu.sync_copy(tab_v, cwk_out.at[pl.ds(off, CWK_BLK)])


def kernel(flatCDK, CWK, CK, flat_eta, N_per_word, flatZ, doc_indexes,
           time_ind_per_word, flatW):
    d32 = doc_indexes.astype(jnp.int32)
    z32 = flatZ.astype(jnp.int32)
    t32 = time_ind_per_word.astype(jnp.int32)
    w32 = flatW.astype(jnp.int32)

    doc_bounds = jnp.arange(0, D_TOTAL + 1, DOCS_PER_TILE, dtype=jnp.int32)
    s_arr = jnp.sum(d32[None, :] < doc_bounds[:, None], axis=1,
                    dtype=jnp.int32)
    s_arr = jnp.concatenate([s_arr, jnp.zeros((15,), jnp.int32)])

    t_bounds = jnp.arange(0, T + 1, dtype=jnp.int32)
    e_arr = jnp.sum(t32[None, :] < t_bounds[:, None], axis=1,
                    dtype=jnp.int32)
    e_arr = jnp.concatenate([e_arr, jnp.zeros((7,), jnp.int32)])

    mesh = plsc.VectorSubcoreMesh(core_axis_name="c", subcore_axis_name="s",
                                  num_cores=NC, num_subcores=NS)

    docs_call = pl.kernel(
        _docs_body,
        out_type=(
            jax.ShapeDtypeStruct((D_TOTAL * K,), jnp.float32),
            jax.ShapeDtypeStruct((D_TOTAL * K,), jnp.float32),
            jax.ShapeDtypeStruct((NC, 16, K), jnp.float32),
            jax.ShapeDtypeStruct((ALL_WORDS,), jnp.int32),
        ),
        mesh=mesh,
        compiler_params=pltpu.CompilerParams(needs_layout_passes=False),
        scratch_types=(
            pltpu.VMEM((48,), jnp.int32),
            pltpu.VMEM((CDK_BLK,), jnp.float32),
            pltpu.VMEM((CDK_BLK,), jnp.float32),
            pltpu.VMEM((16, K), jnp.float32),
            pltpu.VMEM((16,), jnp.int32),
            pltpu.VMEM((CHUNK1,), jnp.int32),
            pltpu.VMEM((CHUNK1,), jnp.int32),
            pltpu.VMEM((CHUNK1,), jnp.int32),
            pltpu.VMEM((CHUNK1,), jnp.float32),
            pltpu.VMEM((CHUNK1,), jnp.int32),
            pltpu.VMEM((CHUNK1,), jnp.int32),
            pltpu.VMEM_SHARED((16, K), jnp.float32),
        ),
    )
    cdk_cnt, eta_cnt, ckp, pk = docs_call(
        d32, z32, t32, N_per_word.astype(jnp.float32), w32, s_arr)

    cwk_call = pl.kernel(
        _cwk_body,
        out_type=jax.ShapeDtypeStruct((T * V * K,), jnp.float32),
        mesh=mesh,
        compiler_params=pltpu.CompilerParams(needs_layout_passes=False),
        scratch_types=(
            pltpu.VMEM((16,), jnp.int32),
            pltpu.VMEM((CWK_BLK,), jnp.float32),
            pltpu.VMEM((CHUNK2,), jnp.int32),
            pltpu.VMEM((CHUNK2,), jnp.int32),
            pltpu.SemaphoreType.DMA,
            pltpu.SemaphoreType.DMA,
        ),
    )
    cwk_cnt = cwk_call(pk, e_arr)

    ck_out = CK + ckp[0, :T, :] + ckp[1, :T, :]
    cdk_out = flatCDK + cdk_cnt.reshape(D_TOTAL, K)
    eta_out = flat_eta + eta_cnt.reshape(D_TOTAL, K)
    cwk_out = CWK + cwk_cnt.reshape(T, V, K)

    return (cdk_out, cwk_out, ck_out, eta_out)
